# baseline (speedup 1.0000x reference)
import functools

import jax
import jax.numpy as jnp
from jax import lax
from jax.experimental import pallas as pl
from jax.experimental.pallas import tpu as pltpu
from jax.experimental.pallas import tpu_sc as plsc

_TAU = 0.9
_TOP_P = 0.8
_TB = 1024
_NW = 32


def _gate_body(p_ref, w_ref, cb_ref, fcw_ref, fcb_ref, o_ref, acc_ref):
    h = pl.program_id(1)
    w = w_ref[...]
    cb = cb_ref[...]

    parts = []
    for v in range(8):
        x = p_ref[:, v, :]
        hc = jax.lax.dot_general(
            w, x, (((1,), (0,)), ((), ())),
            preferred_element_type=jnp.float32)
        parts.append(jnp.maximum(hc + cb, 0.0))
    s8 = ((parts[0] + parts[1]) + (parts[2] + parts[3])) + \
         ((parts[4] + parts[5]) + (parts[6] + parts[7]))

    @pl.when(h == 0)
    def _init():
        acc_ref[...] = s8

    @pl.when(h > 0)
    def _acc():
        acc_ref[...] = acc_ref[...] + s8

    @pl.when(h == pl.num_programs(1) - 1)
    def _finish():
        pooled = acc_ref[...] * (1.0 / 64.0)
        o_ref[...] = jax.lax.dot_general(
            pooled, fcw_ref[...], (((0,), (1,)), ((), ())),
            preferred_element_type=jnp.float32) + fcb_ref[...]


def _route_body(lg_hbm, out_hbm, lg_v, out_v):
    n_tok = out_hbm.shape[0]
    per_w = n_tok // _NW
    wid = lax.axis_index("s") * 2 + lax.axis_index("c")
    base = wid * per_w
    pltpu.sync_copy(lg_hbm.at[pl.ds(base, per_w)], lg_v)

    lanes = lax.iota(jnp.int32, 16)
    _dnums = lax.GatherDimensionNumbers(
        offset_dims=(), collapsed_slice_dims=(0,), start_index_map=(0,))

    def bcast(x, k):
        return lax.gather(
            x, jnp.full((16, 1), k, jnp.int32), _dnums, (1,),
            mode=lax.GatherScatterMode.PROMISE_IN_BOUNDS)

    def tok(t, carry):
        col = lg_v[t]
        srt_l, idx = plsc.sort_key_val(col, lanes, descending=True)
        bmax = bcast(srt_l, 0)
        es = jnp.exp((srt_l - bmax) * (1.0 / _TAU))
        cs = plsc.cumsum(es)
        btot = bcast(cs, 15)
        keep = (cs <= _TOP_P * btot) | (lanes < 1)
        m = jnp.where(keep, es, jnp.float32(0.0))
        bden = bcast(plsc.cumsum(m), 15)
        o = m / bden
        _, o_orig = plsc.sort_key_val(idx, o)
        out_v[t] = o_orig
        return carry

    lax.fori_loop(0, per_w, tok, 0)
    pltpu.sync_copy(out_v, out_hbm.at[pl.ds(base, per_w)])


def kernel(patch, conv_w, conv_b, fc_w, fc_b, layer_idx, threshold):
    del layer_idx, threshold
    n_tok = patch.shape[0]
    q = patch.transpose(1, 2, 3, 0).reshape(196, 64, n_tok)

    n_chunks = n_tok // _TB
    per_w = _TB // _NW
    route = functools.partial(
        pl.kernel,
        mesh=plsc.VectorSubcoreMesh(core_axis_name="c", subcore_axis_name="s"),
        compiler_params=pltpu.CompilerParams(needs_layout_passes=False),
        out_type=jax.ShapeDtypeStruct((_TB, 16), jnp.float32),
        scratch_types=[
            pltpu.VMEM((per_w, 16), jnp.float32),
            pltpu.VMEM((per_w, 16), jnp.float32),
        ],
    )(_route_body)

    outs = []
    for ci in range(n_chunks):
        logits = pl.pallas_call(
            _gate_body,
            grid=(1, 8),
            in_specs=[
                pl.BlockSpec((196, 8, _TB),
                             lambda tb, h, ci=ci: (0, h, ci)),
                pl.BlockSpec((128, 196), lambda tb, h: (0, 0)),
                pl.BlockSpec((128, 1), lambda tb, h: (0, 0)),
                pl.BlockSpec((16, 128), lambda tb, h: (0, 0)),
                pl.BlockSpec((1, 16), lambda tb, h: (0, 0)),
            ],
            out_specs=pl.BlockSpec((_TB, 16), lambda tb, h: (tb, 0)),
            out_shape=jax.ShapeDtypeStruct((_TB, 16), jnp.float32),
            scratch_shapes=[pltpu.VMEM((128, _TB), jnp.float32)],
        )(q, conv_w, conv_b.reshape(128, 1), fc_w, fc_b.reshape(1, 16))
        outs.append(route(logits))
    return jnp.concatenate(outs, axis=0)


# speedup vs baseline: 1.0806x; 1.0806x over previous
import functools

import jax
import jax.numpy as jnp
from jax import lax
from jax.experimental import pallas as pl
from jax.experimental.pallas import tpu as pltpu
from jax.experimental.pallas import tpu_sc as plsc

_TAU = 0.9
_TOP_P = 0.8
_TB = 2048
_NW = 32


def _gate_body(p_ref, w_ref, cb_ref, fcw_ref, fcb_ref, o_ref, acc_ref):
    h = pl.program_id(1)
    w = w_ref[...]
    cb = cb_ref[...]

    parts = []
    for v in range(8):
        x = p_ref[:, v, :]
        hc = jax.lax.dot_general(
            w, x, (((1,), (0,)), ((), ())),
            preferred_element_type=jnp.float32)
        parts.append(jnp.maximum(hc + cb, 0.0))
    s8 = ((parts[0] + parts[1]) + (parts[2] + parts[3])) + \
         ((parts[4] + parts[5]) + (parts[6] + parts[7]))

    @pl.when(h == 0)
    def _init():
        acc_ref[...] = s8

    @pl.when(h > 0)
    def _acc():
        acc_ref[...] = acc_ref[...] + s8

    @pl.when(h == pl.num_programs(1) - 1)
    def _finish():
        pooled = acc_ref[...] * (1.0 / 64.0)
        o_ref[...] = jax.lax.dot_general(
            pooled, fcw_ref[...], (((0,), (1,)), ((), ())),
            preferred_element_type=jnp.float32) + fcb_ref[...]


def _route_body(lg_hbm, out_hbm, lg_v, out_v):
    n_tok = out_hbm.shape[0]
    per_w = n_tok // _NW
    wid = lax.axis_index("s") * 2 + lax.axis_index("c")
    base = wid * per_w
    pltpu.sync_copy(lg_hbm.at[pl.ds(base, per_w)], lg_v)

    lanes = lax.iota(jnp.int32, 16)
    _dnums = lax.GatherDimensionNumbers(
        offset_dims=(), collapsed_slice_dims=(0,), start_index_map=(0,))

    def bcast(x, k):
        return lax.gather(
            x, jnp.full((16, 1), k, jnp.int32), _dnums, (1,),
            mode=lax.GatherScatterMode.PROMISE_IN_BOUNDS)

    def route_one(t):
        col = lg_v[t]
        srt_l, idx = plsc.sort_key_val(col, lanes, descending=True)
        bmax = bcast(srt_l, 0)
        es = jnp.exp((srt_l - bmax) * (1.0 / _TAU))
        cs = plsc.cumsum(es)
        btot = bcast(cs, 15)
        keep = (cs <= _TOP_P * btot) | (lanes < 1)
        m = jnp.where(keep, es, jnp.float32(0.0))
        bden = bcast(plsc.cumsum(m), 15)
        o = m / bden
        _, o_orig = plsc.sort_key_val(idx, o)
        out_v[t] = o_orig

    def tok(u, carry):
        for j in range(4):
            route_one(u * 4 + j)
        return carry

    lax.fori_loop(0, per_w // 4, tok, 0)
    pltpu.sync_copy(out_v, out_hbm.at[pl.ds(base, per_w)])


def kernel(patch, conv_w, conv_b, fc_w, fc_b, layer_idx, threshold):
    del layer_idx, threshold
    n_tok = patch.shape[0]
    q = patch.transpose(1, 2, 3, 0).reshape(196, 64, n_tok)

    grid = (n_tok // _TB, 8)
    logits = pl.pallas_call(
        _gate_body,
        grid=grid,
        in_specs=[
            pl.BlockSpec((196, 8, _TB), lambda tb, h: (0, h, tb)),
            pl.BlockSpec((128, 196), lambda tb, h: (0, 0)),
            pl.BlockSpec((128, 1), lambda tb, h: (0, 0)),
            pl.BlockSpec((16, 128), lambda tb, h: (0, 0)),
            pl.BlockSpec((1, 16), lambda tb, h: (0, 0)),
        ],
        out_specs=pl.BlockSpec((_TB, 16), lambda tb, h: (tb, 0)),
        out_shape=jax.ShapeDtypeStruct((n_tok, 16), jnp.float32),
        scratch_shapes=[pltpu.VMEM((128, _TB), jnp.float32)],
    )(q, conv_w, conv_b.reshape(128, 1), fc_w, fc_b.reshape(1, 16))

    per_w = n_tok // _NW
    route = functools.partial(
        pl.kernel,
        mesh=plsc.VectorSubcoreMesh(core_axis_name="c", subcore_axis_name="s"),
        compiler_params=pltpu.CompilerParams(needs_layout_passes=False),
        out_type=jax.ShapeDtypeStruct((n_tok, 16), jnp.float32),
        scratch_types=[
            pltpu.VMEM((per_w, 16), jnp.float32),
            pltpu.VMEM((per_w, 16), jnp.float32),
        ],
    )(_route_body)
    return route(logits)
